# baseline (speedup 1.0000x reference)
import functools

import jax
import jax.numpy as jnp
from jax import lax
from jax.experimental import pallas as pl
from jax.experimental.pallas import tpu as pltpu
from jax.experimental.pallas import tpu_sc as plsc

_T = 0.3
_L = 128
_NPAD = 20480
_NBLK = _NPAD // _L
_NSUB = 16
_SHARD = _NPAD // _NSUB
_RPT = _NBLK // _NSUB
_NCH = _SHARD // 16


def _sc_body(x1h, y1h, x2h, y2h, outh,
             x1v, y1v, x2v, y2v, keepv, kblkv, pubs):
    sid = lax.axis_index("s")
    cid = lax.axis_index("c")
    iota = lax.broadcasted_iota(jnp.int32, (16,), 0)

    pltpu.sync_copy(x1h, x1v.at[pl.ds(0, _NPAD)])
    pltpu.sync_copy(y1h, y1v.at[pl.ds(0, _NPAD)])
    pltpu.sync_copy(x2h, x2v.at[pl.ds(0, _NPAD)])
    pltpu.sync_copy(y2h, y2v.at[pl.ds(0, _NPAD)])

    def init_body(i, c):
        keepv[pl.ds(i * 16, 16)] = jnp.full((16,), 1.0, jnp.float32)
        return c

    lax.fori_loop(0, _NCH, init_body, 0)

    def splat(ref, g):
        w = ref[pl.ds(g, 16)]
        return jnp.full((16,), w[0], jnp.float32)

    def sup16(bx1, by1, bx2, by2, ba, gb):
        vx1 = x1v[pl.ds(gb, 16)]
        vy1 = y1v[pl.ds(gb, 16)]
        vx2 = x2v[pl.ds(gb, 16)]
        vy2 = y2v[pl.ds(gb, 16)]
        va = (vx2 - vx1) * (vy2 - vy1)
        iw = jnp.maximum(jnp.minimum(bx2, vx2) - jnp.maximum(bx1, vx1), 0.0)
        ih = jnp.maximum(jnp.minimum(by2, vy2) - jnp.maximum(by1, vy1), 0.0)
        inter = iw * ih
        iou = inter / (ba + va - inter)
        return iou > _T

    def blk_body(blk, carry):
        owner = blk // _RPT
        loff = pl.multiple_of(blk * _L - owner * _SHARD, _L)

        @pl.when(sid == owner)
        def _publish():
            pltpu.sync_copy(keepv.at[pl.ds(loff, _L)], pubs)

        plsc.subcore_barrier()
        pltpu.sync_copy(pubs, kblkv.at[pl.ds(0, _L)])

        gblk = blk * _L

        def sg_body(b, c):
            @pl.when(kblkv[pl.ds(b, 16)][0] > 0.0)
            def _s():
                g = gblk + b
                bx1 = splat(x1v, g)
                by1 = splat(y1v, g)
                bx2 = splat(x2v, g)
                by2 = splat(y2v, g)
                ba = (bx2 - bx1) * (by2 - by1)

                def ch_body(ch, c2):
                    base = ch * 16
                    sup = sup16(bx1, by1, bx2, by2, ba, gblk + base)
                    sup = sup & ((iota + base) > b)
                    kv = kblkv[pl.ds(base, 16)]
                    kblkv[pl.ds(base, 16)] = jnp.where(sup, 0.0, kv)
                    return c2

                lax.fori_loop(b // 16, _L // 16, ch_body, 0)
            return c

        lax.fori_loop(0, _L, sg_body, 0)

        shard0 = sid * _SHARD
        lc0 = jnp.maximum(gblk - shard0, 0) // 16

        def fs_body(b, c):
            @pl.when(kblkv[pl.ds(b, 16)][0] > 0.0)
            def _s():
                g = gblk + b
                bx1 = splat(x1v, g)
                by1 = splat(y1v, g)
                bx2 = splat(x2v, g)
                by2 = splat(y2v, g)
                ba = (bx2 - bx1) * (by2 - by1)

                def lc_body(lc, c2):
                    gb = shard0 + lc * 16
                    sup = sup16(bx1, by1, bx2, by2, ba, gb)
                    sup = sup & ((iota + gb) > g)
                    kv = keepv[pl.ds(lc * 16, 16)]
                    keepv[pl.ds(lc * 16, 16)] = jnp.where(sup, 0.0, kv)
                    return c2

                lax.fori_loop(lc0, _NCH, lc_body, 0)
            return c

        lax.fori_loop(0, _L, fs_body, 0)

        plsc.subcore_barrier()
        return carry

    lax.fori_loop(0, _NBLK, blk_body, 0)

    @pl.when(cid == 0)
    def _writeout():
        pltpu.sync_copy(keepv, outh.at[pl.ds(sid * _SHARD, _SHARD)])


@jax.jit
def _sc_nms(x1, y1, x2, y2):
    mesh = plsc.VectorSubcoreMesh(core_axis_name="c", subcore_axis_name="s")
    f = pl.kernel(
        _sc_body,
        mesh=mesh,
        out_type=jax.ShapeDtypeStruct((_NPAD,), jnp.float32),
        scratch_types=[
            pltpu.VMEM((_NPAD + 16,), jnp.float32),
            pltpu.VMEM((_NPAD + 16,), jnp.float32),
            pltpu.VMEM((_NPAD + 16,), jnp.float32),
            pltpu.VMEM((_NPAD + 16,), jnp.float32),
            pltpu.VMEM((_SHARD,), jnp.float32),
            pltpu.VMEM((_L + 16,), jnp.float32),
            pltpu.VMEM_SHARED((_L,), jnp.float32),
        ],
    )
    return f(x1, y1, x2, y2)


def kernel(boxes, scores):
    n = scores.shape[0]
    order = jnp.argsort(-scores)
    bs = jnp.pad(jnp.take(boxes, order, axis=0), ((0, _NPAD - n), (0, 0)))
    keepf = _sc_nms(bs[:, 0], bs[:, 1], bs[:, 2], bs[:, 3])
    keep_sorted = keepf[:n] > 0.0
    keep = jnp.zeros((n,), bool).at[order].set(keep_sorted)
    masked_scores = scores * keep.astype(scores.dtype)
    return masked_scores, keep.astype(jnp.int32)


# speedup vs baseline: 3.2298x; 3.2298x over previous
import jax
import jax.numpy as jnp
from jax import lax
from jax.experimental import pallas as pl
from jax.experimental.pallas import tpu as pltpu
from jax.experimental.pallas import tpu_sc as plsc

_T = 0.3
_L = 128
_NPAD = 20480
_NBLK = _NPAD // _L
_NSUB = 16
_SHARD = _NPAD // _NSUB
_NCH = _SHARD // 16
_G = 4


def _sc_body(x1h, y1h, x2h, y2h, outh,
             x1v, y1v, x2v, y2v, keepv, kblkv,
             kx1b, ky1b, kx2b, ky2b, kab, kib, nkc, pubs):
    sid = lax.axis_index("s")
    cid = lax.axis_index("c")
    iota = lax.broadcasted_iota(jnp.int32, (16,), 0)
    ones16 = jnp.full((16,), 1.0, jnp.float32)
    zero16 = jnp.zeros((16,), jnp.float32)

    pltpu.sync_copy(x1h, x1v.at[pl.ds(0, _NPAD)])
    pltpu.sync_copy(y1h, y1v.at[pl.ds(0, _NPAD)])
    pltpu.sync_copy(x2h, x2v.at[pl.ds(0, _NPAD)])
    pltpu.sync_copy(y2h, y2v.at[pl.ds(0, _NPAD)])

    def init_body(i, c):
        keepv[pl.ds(i * 16, 16)] = ones16
        return c

    lax.fori_loop(0, _NCH, init_body, 0)

    x1v[pl.ds(_NPAD, 16)] = zero16
    y1v[pl.ds(_NPAD, 16)] = zero16
    x2v[pl.ds(_NPAD, 16)] = zero16
    y2v[pl.ds(_NPAD, 16)] = zero16

    def splat(ref, g):
        w = ref[pl.ds(g, 16)]
        return jnp.full((16,), w[0], jnp.float32)

    def blk_body(blk, carry):
        g0 = blk * (_L // 16)
        for j in range(_L // 16):
            gc = g0 + j

            @pl.when(sid == lax.rem(gc, _NSUB))
            def _publish():
                lo = pl.multiple_of((gc // _NSUB) * 16, 16)
                pltpu.sync_copy(keepv.at[pl.ds(lo, 16)],
                                pubs.at[pl.ds(j * 16, 16)])

        plsc.subcore_barrier()
        pltpu.sync_copy(pubs, kblkv.at[pl.ds(0, _L)])

        gblk = blk * _L
        nkc[pl.ds(0, 16)] = jnp.zeros((16,), jnp.int32)

        def sg_body(b, c):
            @pl.when(kblkv[pl.ds(b, 16)][0] > 0.0)
            def _s():
                g = gblk + b
                bx1 = splat(x1v, g)
                by1 = splat(y1v, g)
                bx2 = splat(x2v, g)
                by2 = splat(y2v, g)
                ba = (bx2 - bx1) * (by2 - by1)

                nk = nkc[pl.ds(0, 16)][0]
                off = pl.multiple_of(nk * 16, 16)
                kx1b[pl.ds(off, 16)] = bx1
                ky1b[pl.ds(off, 16)] = by1
                kx2b[pl.ds(off, 16)] = bx2
                ky2b[pl.ds(off, 16)] = by2
                kab[pl.ds(off, 16)] = ba
                kib[pl.ds(off, 16)] = jnp.full((16,), g, jnp.int32)
                nkc[pl.ds(0, 16)] = jnp.full((16,), nk + 1, jnp.int32)

                def ch_body(ch, c2):
                    base = ch * 16
                    gb = gblk + base
                    vx1 = x1v[pl.ds(gb, 16)]
                    vy1 = y1v[pl.ds(gb, 16)]
                    vx2 = x2v[pl.ds(gb, 16)]
                    vy2 = y2v[pl.ds(gb, 16)]
                    va = (vx2 - vx1) * (vy2 - vy1)
                    iw = jnp.maximum(
                        jnp.minimum(bx2, vx2) - jnp.maximum(bx1, vx1), 0.0)
                    ih = jnp.maximum(
                        jnp.minimum(by2, vy2) - jnp.maximum(by1, vy1), 0.0)
                    inter = iw * ih
                    iou = inter / (ba + va - inter)
                    sup = (iou > _T) & ((iota + base) > b)
                    kv = kblkv[pl.ds(base, 16)]
                    kblkv[pl.ds(base, 16)] = jnp.where(sup, 0.0, kv)
                    return c2

                lax.fori_loop(b // 16, _L // 16, ch_body, 0)
            return c

        lax.fori_loop(0, _L, sg_body, 0)

        nk = nkc[pl.ds(0, 16)][0]
        for d in range(_G - 1):
            off = pl.multiple_of((nk + d) * 16, 16)
            kx1b[pl.ds(off, 16)] = zero16
            ky1b[pl.ds(off, 16)] = zero16
            kx2b[pl.ds(off, 16)] = zero16
            ky2b[pl.ds(off, 16)] = zero16
            kab[pl.ds(off, 16)] = zero16
            kib[pl.ds(off, 16)] = jnp.full((16,), _NPAD, jnp.int32)

        lc0 = (blk * (_L // 16) - sid + (_NSUB - 1)) // _NSUB

        def grp_body(it, c):
            sups = []
            for j in range(_G):
                base = pl.multiple_of((it * _G + j) * 16, 16)
                sups.append((kx1b[pl.ds(base, 16)],
                             ky1b[pl.ds(base, 16)],
                             kx2b[pl.ds(base, 16)],
                             ky2b[pl.ds(base, 16)],
                             kab[pl.ds(base, 16)],
                             kib[pl.ds(base, 16)]))

            def lc_body(lc, c2):
                gb = (lc * _NSUB + sid) * 16
                vx1 = x1v[pl.ds(gb, 16)]
                vy1 = y1v[pl.ds(gb, 16)]
                vx2 = x2v[pl.ds(gb, 16)]
                vy2 = y2v[pl.ds(gb, 16)]
                va = (vx2 - vx1) * (vy2 - vy1)
                lanes = iota + gb
                supf = zero16
                for j in range(_G):
                    bx1, by1, bx2, by2, ba, gv = sups[j]
                    iw = jnp.maximum(
                        jnp.minimum(bx2, vx2) - jnp.maximum(bx1, vx1), 0.0)
                    ih = jnp.maximum(
                        jnp.minimum(by2, vy2) - jnp.maximum(by1, vy1), 0.0)
                    inter = iw * ih
                    iou = inter / (ba + va - inter)
                    supf = jnp.maximum(
                        supf,
                        jnp.where((iou > _T) & (lanes > gv), 1.0, 0.0))
                kv = keepv[pl.ds(lc * 16, 16)]
                keepv[pl.ds(lc * 16, 16)] = jnp.where(supf > 0.0, 0.0, kv)
                return c2

            lax.fori_loop(lc0, _NCH, lc_body, 0)
            return c

        lax.fori_loop(0, (nk + _G - 1) // _G, grp_body, 0)

        plsc.subcore_barrier()
        return carry

    lax.fori_loop(0, _NBLK, blk_body, 0)

    @pl.when(cid == 0)
    def _writeout():
        pltpu.sync_copy(keepv, outh.at[pl.ds(sid * _SHARD, _SHARD)])


@jax.jit
def _sc_nms(x1, y1, x2, y2):
    mesh = plsc.VectorSubcoreMesh(core_axis_name="c", subcore_axis_name="s")
    nstage = (_L + _G) * 16
    f = pl.kernel(
        _sc_body,
        mesh=mesh,
        out_type=jax.ShapeDtypeStruct((_NPAD,), jnp.float32),
        scratch_types=[
            pltpu.VMEM((_NPAD + 16,), jnp.float32),
            pltpu.VMEM((_NPAD + 16,), jnp.float32),
            pltpu.VMEM((_NPAD + 16,), jnp.float32),
            pltpu.VMEM((_NPAD + 16,), jnp.float32),
            pltpu.VMEM((_SHARD,), jnp.float32),
            pltpu.VMEM((_L + 16,), jnp.float32),
            pltpu.VMEM((nstage,), jnp.float32),
            pltpu.VMEM((nstage,), jnp.float32),
            pltpu.VMEM((nstage,), jnp.float32),
            pltpu.VMEM((nstage,), jnp.float32),
            pltpu.VMEM((nstage,), jnp.float32),
            pltpu.VMEM((nstage,), jnp.int32),
            pltpu.VMEM((16,), jnp.int32),
            pltpu.VMEM_SHARED((_L,), jnp.float32),
        ],
    )
    return f(x1, y1, x2, y2)


def kernel(boxes, scores):
    n = scores.shape[0]
    order = jnp.argsort(-scores)
    bs = jnp.pad(jnp.take(boxes, order, axis=0), ((0, _NPAD - n), (0, 0)))
    keepf = _sc_nms(bs[:, 0], bs[:, 1], bs[:, 2], bs[:, 3])
    keepf = keepf.reshape(_NSUB, _NCH, 16).transpose(1, 0, 2).reshape(-1)
    keep_sorted = keepf[:n] > 0.0
    keep = jnp.zeros((n,), bool).at[order].set(keep_sorted)
    masked_scores = scores * keep.astype(scores.dtype)
    return masked_scores, keep.astype(jnp.int32)
